# Initial kernel scaffold; baseline (speedup 1.0000x reference)
import jax
import jax.numpy as jnp
from jax.experimental import pallas as pl


def kernel(features, edge_index, W1, b1, W2, b2):
    raise NotImplementedError("write your pallas kernel here")


# speedup vs baseline: 13.0834x; 13.0834x over previous
import functools

import jax
import jax.numpy as jnp
from jax import lax
from jax.experimental import pallas as pl
from jax.experimental.pallas import tpu as pltpu
from jax.experimental.pallas import tpu_sc as plsc

N_NODES = 10000
N_EDGES = 320000
IN_FEATS = 128
N_HIDDEN = 16
N_CLASSES = 16

NC = 2
NS = 16
NW = NC * NS
CH = 128

NCH = -(-N_EDGES // (NW * CH))
E_PT = NCH * CH
EPAD = NW * E_PT
NPAD = -(-(N_NODES + 1) // (NS * 8)) * (NS * 8)
ROWS_PT = NPAD // NS


def _sc_edge_pass(table, src3, dst3):
  mesh = plsc.VectorSubcoreMesh(core_axis_name="c", subcore_axis_name="s")

  @functools.partial(
      pl.kernel,
      mesh=mesh,
      compiler_params=pltpu.CompilerParams(use_tc_tiling_on_sc=False),
      out_type=jax.ShapeDtypeStruct((NC, NPAD, N_HIDDEN), jnp.float32),
      scratch_types=[
          pltpu.VMEM((NCH, CH), jnp.int32),
          pltpu.VMEM((NCH, CH), jnp.int32),
          pltpu.VMEM((CH, N_HIDDEN), jnp.float32),
          pltpu.VMEM((ROWS_PT, N_HIDDEN), jnp.float32),
          pltpu.VMEM_SHARED((NPAD, N_HIDDEN), jnp.float32),
          pltpu.SemaphoreType.DMA,
      ],
  )
  def edge_pass(table_hbm, src_hbm, dst_hbm, out_hbm,
                src_v, dst_v, rows_v, zero_v, accum_sh, sem):
    c = lax.axis_index("c")
    s = lax.axis_index("s")
    wid = s * NC + c

    def zbody(i, carry):
      zero_v[i] = jnp.zeros((N_HIDDEN,), jnp.float32)
      return carry
    lax.fori_loop(0, ROWS_PT, zbody, 0)
    pltpu.sync_copy(zero_v, accum_sh.at[pl.ds(s * ROWS_PT, ROWS_PT)])

    pltpu.sync_copy(src_hbm.at[wid], src_v)
    pltpu.sync_copy(dst_hbm.at[wid], dst_v)
    plsc.subcore_barrier()

    def chunk(j, carry):
      pltpu.async_copy(table_hbm.at[src_v.at[j]], rows_v, sem).wait()
      pltpu.sync_copy(rows_v, accum_sh.at[dst_v.at[j]], add=True)
      return carry
    lax.fori_loop(0, NCH, chunk, 0)

    plsc.subcore_barrier()
    pltpu.sync_copy(accum_sh.at[pl.ds(s * ROWS_PT, ROWS_PT)],
                    out_hbm.at[c, pl.ds(s * ROWS_PT, ROWS_PT)])

  return edge_pass(table, src3, dst3)


def _tc_project1(features, w1t):
  def body(x_ref, w_ref, o_ref):
    o_ref[...] = jnp.dot(x_ref[...], w_ref[...],
                         preferred_element_type=jnp.float32)
  return pl.pallas_call(
      body,
      out_shape=jax.ShapeDtypeStruct((N_NODES, N_HIDDEN), jnp.float32),
  )(features, w1t)


def _tc_mid(partials, b1, w2t):
  def body(p_ref, b_ref, w_ref, o_ref):
    h = p_ref[0] + p_ref[1] + b_ref[...]
    h = jnp.maximum(h, 0.0)
    o_ref[...] = jnp.dot(h, w_ref[...], preferred_element_type=jnp.float32)
  return pl.pallas_call(
      body,
      out_shape=jax.ShapeDtypeStruct((NPAD, N_HIDDEN), jnp.float32),
  )(partials, b1, w2t)


def _tc_final(partials, b2):
  def body(p_ref, b_ref, o_ref):
    o_ref[...] = p_ref[0, :N_NODES] + p_ref[1, :N_NODES] + b_ref[...]
  return pl.pallas_call(
      body,
      out_shape=jax.ShapeDtypeStruct((N_NODES, N_CLASSES), jnp.float32),
  )(partials, b2)


def kernel(features, edge_index, W1, b1, W2, b2):
  src = edge_index[0].astype(jnp.int32)
  dst = edge_index[1].astype(jnp.int32)
  src3 = jnp.concatenate(
      [src, jnp.zeros((EPAD - N_EDGES,), jnp.int32)]).reshape(NW, NCH, CH)
  dst3 = jnp.concatenate(
      [dst, jnp.full((EPAD - N_EDGES,), N_NODES, jnp.int32)]
  ).reshape(NW, NCH, CH)

  h1pre = _tc_project1(features, W1.T)
  parts1 = _sc_edge_pass(h1pre, src3, dst3)
  h2pre = _tc_mid(parts1, b1.reshape(1, N_HIDDEN), W2.T)
  parts2 = _sc_edge_pass(h2pre, src3, dst3)
  return _tc_final(parts2, b2.reshape(1, N_CLASSES))


# speedup vs baseline: 16.7215x; 1.2781x over previous
import functools

import jax
import jax.numpy as jnp
from jax import lax
from jax.experimental import pallas as pl
from jax.experimental.pallas import tpu as pltpu
from jax.experimental.pallas import tpu_sc as plsc

N_NODES = 10000
N_EDGES = 320000
IN_FEATS = 128
N_HIDDEN = 16
N_CLASSES = 16

NC = 2
NS = 16
NW = NC * NS
CH = 128
GC = 512
SPG = GC // CH

NG = -(-N_EDGES // (NW * GC))
E_PT = NG * GC
EPAD = NW * E_PT
NCH = E_PT // CH

NPAD = -(-(N_NODES + 1) // (NS * 8)) * (NS * 8)
ROWS_PT = NPAD // NS


def _sc_edge_pass(table, src3, dst3, zeros):
  mesh = plsc.VectorSubcoreMesh(core_axis_name="c", subcore_axis_name="s")

  @functools.partial(
      pl.kernel,
      mesh=mesh,
      compiler_params=pltpu.CompilerParams(use_tc_tiling_on_sc=False),
      out_type=jax.ShapeDtypeStruct((NC, NPAD, N_HIDDEN), jnp.float32),
      scratch_types=[
          pltpu.VMEM((NG, GC), jnp.int32),
          pltpu.VMEM((NCH, CH), jnp.int32),
          pltpu.VMEM((2, GC, N_HIDDEN), jnp.float32),
          pltpu.VMEM_SHARED((NPAD, N_HIDDEN), jnp.float32),
          pltpu.SemaphoreType.DMA,
          pltpu.SemaphoreType.DMA,
          pltpu.SemaphoreType.DMA,
      ],
  )
  def edge_pass(table_hbm, src_hbm, dst_hbm, zeros_hbm, out_hbm,
                src_v, dst_v, rows_v, accum_sh, gsem, ssem, zsem):
    c = lax.axis_index("c")
    s = lax.axis_index("s")
    wid = s * NC + c
    acc_rows = pl.ds(s * ROWS_PT, ROWS_PT)

    zc = pltpu.async_copy(zeros_hbm.at[acc_rows], accum_sh.at[acc_rows], zsem)
    sc_ = pltpu.async_copy(src_hbm.at[wid], src_v, gsem)
    dc = pltpu.async_copy(dst_hbm.at[wid], dst_v, zsem)
    sc_.wait()
    dc.wait()
    zc.wait()
    plsc.subcore_barrier()

    pltpu.async_copy(table_hbm.at[src_v.at[0]], rows_v.at[0], gsem)

    def body(g, carry):
      bsel = lax.rem(g, 2)
      prev = lax.rem(g + 1, 2)
      pltpu.make_async_copy(
          table_hbm.at[src_v.at[g]], rows_v.at[bsel], gsem).wait()
      @pl.when(g > 0)
      def _():
        for t in range(SPG):
          pltpu.make_async_copy(
              rows_v.at[prev, pl.ds(t * CH, CH)],
              accum_sh.at[dst_v.at[(g - 1) * SPG + t]], ssem).wait()
      @pl.when(g + 1 < NG)
      def _():
        pltpu.async_copy(
            table_hbm.at[src_v.at[g + 1]], rows_v.at[prev], gsem)
      for t in range(SPG):
        pltpu.async_copy(
            rows_v.at[bsel, pl.ds(t * CH, CH)],
            accum_sh.at[dst_v.at[g * SPG + t]], ssem, add=True)
      return carry

    lax.fori_loop(0, NG, body, 0)

    last = (NG - 1) % 2
    for t in range(SPG):
      pltpu.make_async_copy(
          rows_v.at[last, pl.ds(t * CH, CH)],
          accum_sh.at[dst_v.at[(NG - 1) * SPG + t]], ssem).wait()

    plsc.subcore_barrier()
    pltpu.sync_copy(accum_sh.at[acc_rows], out_hbm.at[c, acc_rows])

  return edge_pass(table, src3, dst3, zeros)


def _tc_project1(features, w1t):
  def body(x_ref, w_ref, o_ref):
    o_ref[...] = jnp.dot(x_ref[...], w_ref[...],
                         preferred_element_type=jnp.float32)
  return pl.pallas_call(
      body,
      out_shape=jax.ShapeDtypeStruct((N_NODES, N_HIDDEN), jnp.float32),
  )(features, w1t)


def _tc_mid(partials, b1, w2t):
  def body(p_ref, b_ref, w_ref, o_ref):
    h = p_ref[0] + p_ref[1] + b_ref[...]
    h = jnp.maximum(h, 0.0)
    o_ref[...] = jnp.dot(h, w_ref[...], preferred_element_type=jnp.float32)
  return pl.pallas_call(
      body,
      out_shape=jax.ShapeDtypeStruct((NPAD, N_HIDDEN), jnp.float32),
  )(partials, b1, w2t)


def _tc_final(partials, b2):
  def body(p_ref, b_ref, o_ref):
    o_ref[...] = p_ref[0, :N_NODES] + p_ref[1, :N_NODES] + b_ref[...]
  return pl.pallas_call(
      body,
      out_shape=jax.ShapeDtypeStruct((N_NODES, N_CLASSES), jnp.float32),
  )(partials, b2)


def kernel(features, edge_index, W1, b1, W2, b2):
  src = edge_index[0].astype(jnp.int32)
  dst = edge_index[1].astype(jnp.int32)
  src3 = jnp.concatenate(
      [src, jnp.zeros((EPAD - N_EDGES,), jnp.int32)]).reshape(NW, NG, GC)
  dst3 = jnp.concatenate(
      [dst, jnp.full((EPAD - N_EDGES,), N_NODES, jnp.int32)]
  ).reshape(NW, NCH, CH)
  zeros = jnp.zeros((NPAD, N_HIDDEN), jnp.float32)

  h1pre = _tc_project1(features, W1.T)
  parts1 = _sc_edge_pass(h1pre, src3, dst3, zeros)
  h2pre = _tc_mid(parts1, b1.reshape(1, N_HIDDEN), W2.T)
  parts2 = _sc_edge_pass(h2pre, src3, dst3, zeros)
  return _tc_final(parts2, b2.reshape(1, N_CLASSES))


# speedup vs baseline: 17.8791x; 1.0692x over previous
import functools

import jax
import jax.numpy as jnp
from jax import lax
from jax.experimental import pallas as pl
from jax.experimental.pallas import tpu as pltpu
from jax.experimental.pallas import tpu_sc as plsc

N_NODES = 10000
N_EDGES = 320000
IN_FEATS = 128
N_HIDDEN = 16
N_CLASSES = 16

NC = 2
NS = 16
NW = NC * NS
CH = 128
GC = 512
SPG = GC // CH

NG = -(-N_EDGES // (NW * GC))
E_PT = NG * GC
EPAD = NW * E_PT
NCH = E_PT // CH

NPAD = -(-(N_NODES + 1) // (NS * 8)) * (NS * 8)
ROWS_PT = NPAD // NS


def _sc_edge_pass(table, src3, dst3, zeros):
  mesh = plsc.VectorSubcoreMesh(core_axis_name="c", subcore_axis_name="s")

  @functools.partial(
      pl.kernel,
      mesh=mesh,
      compiler_params=pltpu.CompilerParams(use_tc_tiling_on_sc=False),
      out_type=jax.ShapeDtypeStruct((NC, NPAD, N_HIDDEN), jnp.float32),
      scratch_types=[
          pltpu.VMEM((NG, GC), jnp.int32),
          pltpu.VMEM((NCH, CH), jnp.int32),
          pltpu.VMEM((3, GC, N_HIDDEN), jnp.float32),
          pltpu.VMEM_SHARED((NPAD, N_HIDDEN), jnp.float32),
          pltpu.SemaphoreType.DMA,
          pltpu.SemaphoreType.DMA,
          pltpu.SemaphoreType.DMA,
      ],
  )
  def edge_pass(table_hbm, src_hbm, dst_hbm, zeros_hbm, out_hbm,
                src_v, dst_v, rows_v, accum_sh, gsem, ssem, zsem):
    c = lax.axis_index("c")
    s = lax.axis_index("s")
    wid = s * NC + c
    acc_rows = pl.ds(s * ROWS_PT, ROWS_PT)

    zc = pltpu.async_copy(zeros_hbm.at[acc_rows], accum_sh.at[acc_rows], zsem)
    sc_ = pltpu.async_copy(src_hbm.at[wid], src_v, gsem)
    dc = pltpu.async_copy(dst_hbm.at[wid], dst_v, zsem)
    sc_.wait()
    dc.wait()
    zc.wait()
    plsc.subcore_barrier()

    pltpu.async_copy(table_hbm.at[src_v.at[0]], rows_v.at[0], gsem)
    pltpu.async_copy(table_hbm.at[src_v.at[1]], rows_v.at[1], gsem)

    def body(g, carry):
      bsel = lax.rem(g, 3)
      prev = lax.rem(g + 2, 3)
      pltpu.make_async_copy(
          table_hbm.at[src_v.at[g]], rows_v.at[bsel], gsem).wait()
      @pl.when(g > 0)
      def _():
        for t in range(SPG):
          pltpu.make_async_copy(
              rows_v.at[prev, pl.ds(t * CH, CH)],
              accum_sh.at[dst_v.at[(g - 1) * SPG + t]], ssem).wait()
      @pl.when(g + 2 < NG)
      def _():
        pltpu.async_copy(
            table_hbm.at[src_v.at[g + 2]], rows_v.at[prev], gsem)
      for t in range(SPG):
        pltpu.async_copy(
            rows_v.at[bsel, pl.ds(t * CH, CH)],
            accum_sh.at[dst_v.at[g * SPG + t]], ssem, add=True)
      return carry

    lax.fori_loop(0, NG, body, 0)

    last = (NG - 1) % 3
    for t in range(SPG):
      pltpu.make_async_copy(
          rows_v.at[last, pl.ds(t * CH, CH)],
          accum_sh.at[dst_v.at[(NG - 1) * SPG + t]], ssem).wait()

    plsc.subcore_barrier()
    pltpu.sync_copy(accum_sh.at[acc_rows], out_hbm.at[c, acc_rows])

  return edge_pass(table, src3, dst3, zeros)


def _tc_project1(features, w1t):
  def body(x_ref, w_ref, o_ref):
    o_ref[...] = jnp.dot(x_ref[...], w_ref[...],
                         preferred_element_type=jnp.float32)
  return pl.pallas_call(
      body,
      out_shape=jax.ShapeDtypeStruct((N_NODES, N_HIDDEN), jnp.float32),
  )(features, w1t)


def _tc_mid(partials, b1, w2t):
  def body(p_ref, b_ref, w_ref, o_ref):
    h = p_ref[0] + p_ref[1] + b_ref[...]
    h = jnp.maximum(h, 0.0)
    o_ref[...] = jnp.dot(h, w_ref[...], preferred_element_type=jnp.float32)
  return pl.pallas_call(
      body,
      out_shape=jax.ShapeDtypeStruct((NPAD, N_HIDDEN), jnp.float32),
  )(partials, b1, w2t)


def _tc_final(partials, b2):
  def body(p_ref, b_ref, o_ref):
    o_ref[...] = p_ref[0, :N_NODES] + p_ref[1, :N_NODES] + b_ref[...]
  return pl.pallas_call(
      body,
      out_shape=jax.ShapeDtypeStruct((N_NODES, N_CLASSES), jnp.float32),
  )(partials, b2)


def kernel(features, edge_index, W1, b1, W2, b2):
  src = edge_index[0].astype(jnp.int32)
  dst = edge_index[1].astype(jnp.int32)
  src3 = jnp.concatenate(
      [src, jnp.zeros((EPAD - N_EDGES,), jnp.int32)]).reshape(NW, NG, GC)
  dst3 = jnp.concatenate(
      [dst, jnp.full((EPAD - N_EDGES,), N_NODES, jnp.int32)]
  ).reshape(NW, NCH, CH)
  zeros = jnp.zeros((NPAD, N_HIDDEN), jnp.float32)

  h1pre = _tc_project1(features, W1.T)
  parts1 = _sc_edge_pass(h1pre, src3, dst3, zeros)
  h2pre = _tc_mid(parts1, b1.reshape(1, N_HIDDEN), W2.T)
  parts2 = _sc_edge_pass(h2pre, src3, dst3, zeros)
  return _tc_final(parts2, b2.reshape(1, N_CLASSES))
